# Initial kernel scaffold; baseline (speedup 1.0000x reference)
import jax
import jax.numpy as jnp
from jax.experimental import pallas as pl


def kernel(x, table, W, b):
    raise NotImplementedError("write your pallas kernel here")


# speedup vs baseline: 2.3366x; 2.3366x over previous
import functools

import jax
import jax.numpy as jnp
from jax import lax
from jax.experimental import pallas as pl
from jax.experimental.pallas import tpu as pltpu
from jax.experimental.pallas import tpu_sc as plsc

VOCAB = 100000
EMBED = 64
NE = 50
N_TOK = 4096 * 50

NC, NS = 2, 16
NW = NC * NS
ROWS_PER_W = N_TOK // NW
CHUNK = 128
NCHUNK = ROWS_PER_W // CHUNK
NBUF = 4

_PROJ_BLK = 4000


def _proj_body(table_ref, w_ref, b_ref, out_ref):
    out_ref[...] = (
        jnp.dot(table_ref[...], w_ref[...], preferred_element_type=jnp.float32)
        + b_ref[0:1, :]
    )


def _project_table(table, W, b2d):
    return pl.pallas_call(
        _proj_body,
        grid=(VOCAB // _PROJ_BLK,),
        in_specs=[
            pl.BlockSpec((_PROJ_BLK, EMBED), lambda i: (i, 0)),
            pl.BlockSpec((EMBED, NE), lambda i: (0, 0)),
            pl.BlockSpec((8, NE), lambda i: (0, 0)),
        ],
        out_specs=pl.BlockSpec((_PROJ_BLK, NE), lambda i: (i, 0)),
        out_shape=jax.ShapeDtypeStruct((VOCAB, NE), jnp.float32),
    )(table, W, b2d)


def _gather_body(tp_hbm, idx_hbm, out_hbm, idx_v, bufs, gsems):
    wid = lax.axis_index("s") * NC + lax.axis_index("c")
    base = wid * ROWS_PER_W
    pltpu.sync_copy(idx_hbm.at[wid], idx_v)

    def _issue(c, b):
        return pltpu.async_copy(tp_hbm.at[idx_v.at[c]], bufs[b], gsems[b])

    def _wait(b):
        pltpu.make_async_copy(tp_hbm.at[idx_v.at[0]], bufs[b], gsems[b]).wait()

    for b in range(NBUF):
        _issue(b, b)

    @pl.loop(0, NCHUNK, step=NBUF)
    def _chunk_loop(g):
        for b in range(NBUF):
            c = g + b
            _wait(b)
            pltpu.sync_copy(bufs[b], out_hbm.at[pl.ds(base + c * CHUNK, CHUNK)])
            _issue(jnp.minimum(c + NBUF, NCHUNK - 1), b)

    for b in range(NBUF):
        _wait(b)


@functools.partial(
    pl.kernel,
    out_type=jax.ShapeDtypeStruct((N_TOK, NE), jnp.float32),
    mesh=plsc.VectorSubcoreMesh(core_axis_name="c", subcore_axis_name="s"),
    scratch_types=[
        pltpu.VMEM((NCHUNK, CHUNK), jnp.int32),
    ]
    + [pltpu.VMEM((CHUNK, NE), jnp.float32) for _ in range(NBUF)]
    + [pltpu.SemaphoreType.DMA for _ in range(NBUF)],
    compiler_params=pltpu.CompilerParams(use_tc_tiling_on_sc=False),
)
def _sc_gather(tp_hbm, idx_hbm, out_hbm, idx_v, *rest):
    bufs = rest[:NBUF]
    gsems = rest[NBUF : 2 * NBUF]
    _gather_body(tp_hbm, idx_hbm, out_hbm, idx_v, bufs, gsems)


def kernel(x, table, W, b):
    tp = _project_table(table, W, jnp.broadcast_to(b.reshape(1, NE), (8, NE)))
    idx = x.astype(jnp.int32).reshape(NW, NCHUNK, CHUNK)
    out = _sc_gather(tp, idx)
    return out.reshape(x.shape[0], x.shape[1], NE)
